# baseline (speedup 1.0000x reference)
import functools

import jax
import jax.numpy as jnp
from jax import lax
from jax.experimental import pallas as pl
from jax.experimental.pallas import tpu as pltpu
from jax.experimental.pallas import tpu_sc as plsc

B, C_IN, C_OUT, E, K = 1, 64, 64, 320000, 5
EPS = 1e-05

NC, NS = 2, 16
NW = NC * NS
ROWS_PER_W = K * E // NW
CHUNK = 128
NB = 6
NFULL = ROWS_PER_W // CHUNK
NGROUPS = NFULL // NB
TAIL = ROWS_PER_W - NFULL * CHUNK


def _sc_gather(feT, idx_flat):
    mesh = plsc.VectorSubcoreMesh(core_axis_name="c", subcore_axis_name="s")

    @functools.partial(
        pl.kernel,
        out_type=jax.ShapeDtypeStruct((K * E, C_IN), feT.dtype),
        mesh=mesh,
        scratch_types=[
            pltpu.VMEM((NB, CHUNK), jnp.int32),
            pltpu.VMEM((NB, CHUNK, C_IN), feT.dtype),
            pltpu.SemaphoreType.DMA,
            pltpu.SemaphoreType.DMA,
        ],
        compiler_params=pltpu.CompilerParams(use_tc_tiling_on_sc=False),
    )
    def k(feT_hbm, idx_hbm, out_hbm, idx_v, rows_v, gsem, wsem):
        wid = lax.axis_index("s") * NC + lax.axis_index("c")
        base = wid * ROWS_PER_W

        def group(g, carry):
            starts = [base + (g * NB + b) * CHUNK for b in range(NB)]
            gh = []
            for b in range(NB):
                pltpu.sync_copy(idx_hbm.at[pl.ds(starts[b], CHUNK)], idx_v.at[b])
                gh.append(pltpu.async_copy(feT_hbm.at[idx_v.at[b]], rows_v.at[b], gsem))
            wh = []
            for b in range(NB):
                gh[b].wait()
                wh.append(pltpu.async_copy(rows_v.at[b], out_hbm.at[pl.ds(starts[b], CHUNK)], wsem))
            for b in range(NB):
                wh[b].wait()
            return carry

        lax.fori_loop(0, NGROUPS, group, 0, unroll=False)

        tstart = base + NFULL * CHUNK
        pltpu.sync_copy(idx_hbm.at[pl.ds(tstart, TAIL)], idx_v.at[0, pl.ds(0, TAIL)])
        pltpu.async_copy(feT_hbm.at[idx_v.at[0, pl.ds(0, TAIL)]],
                         rows_v.at[0, pl.ds(0, TAIL)], gsem).wait()
        pltpu.async_copy(rows_v.at[0, pl.ds(0, TAIL)],
                         out_hbm.at[pl.ds(tstart, TAIL)], wsem).wait()

    return k(feT, idx_flat)


ET = 1280


def _tc_transpose(fe2):
    def body(fe_ref, o_ref):
        o_ref[...] = fe_ref[...].T.astype(jnp.bfloat16)

    return pl.pallas_call(
        body,
        grid=(E // ET,),
        in_specs=[pl.BlockSpec((C_IN, ET), lambda i: (0, i))],
        out_specs=pl.BlockSpec((ET, C_IN), lambda i: (i, 0)),
        out_shape=jax.ShapeDtypeStruct((E, C_IN), jnp.bfloat16),
    )(fe2)


def _tc_conv(fg, wmat):
    def body(fg_ref, w_ref, raw_ref, stats_ref):
        i = pl.program_id(0)
        f0 = fg_ref[0]
        f1 = fg_ref[1]
        f2 = fg_ref[2]
        f3 = fg_ref[3]
        f4 = fg_ref[4]
        fn = jnp.concatenate(
            [f0, f1 + f3, f2 + f4, jnp.abs(f1 - f3), jnp.abs(f2 - f4)], axis=1)
        out = lax.dot_general(w_ref[...], fn, (((1,), (1,)), ((), ())),
                              preferred_element_type=jnp.float32)
        raw_ref[...] = out.astype(jnp.bfloat16)

        @pl.when(i == 0)
        def _init():
            stats_ref[...] = jnp.zeros_like(stats_ref)

        s = jnp.sum(out, axis=1, keepdims=True)
        sq = jnp.sum(out * out, axis=1, keepdims=True)
        stats_ref[...] += jnp.concatenate([s, sq], axis=1)

    return pl.pallas_call(
        body,
        grid=(E // ET,),
        in_specs=[pl.BlockSpec((K, ET, C_IN), lambda i: (0, i, 0)),
                  pl.BlockSpec((C_OUT, K * C_IN), lambda i: (0, 0))],
        out_specs=[pl.BlockSpec((C_OUT, ET), lambda i: (0, i)),
                   pl.BlockSpec((C_OUT, 2), lambda i: (0, 0))],
        out_shape=[jax.ShapeDtypeStruct((C_OUT, E), jnp.bfloat16),
                   jax.ShapeDtypeStruct((C_OUT, 2), jnp.float32)],
    )(fg, wmat)


def _tc_norm(raw, stats):
    inv_e = 1.0 / E

    def body(raw_ref, st_ref, o_ref):
        st = st_ref[...]
        mean = st[:, 0:1] * inv_e
        ex2 = st[:, 1:2] * inv_e
        var = jnp.maximum(ex2 - mean * mean, 0.0)
        rstd = 1.0 / (jnp.sqrt(var) + EPS)
        raw = raw_ref[...].astype(jnp.float32)
        o_ref[...] = jnp.maximum((raw - mean) * rstd, 0.0)

    return pl.pallas_call(
        body,
        grid=(E // ET,),
        in_specs=[pl.BlockSpec((C_OUT, ET), lambda i: (0, i)),
                  pl.BlockSpec((C_OUT, 2), lambda i: (0, 0))],
        out_specs=pl.BlockSpec((C_OUT, ET), lambda i: (0, i)),
        out_shape=jax.ShapeDtypeStruct((C_OUT, E), jnp.float32),
    )(raw, stats)


def kernel(fe, gemm_edges, W, b):
    del b
    fe2 = fe[0]
    idx_flat = gemm_edges[0].T.reshape(K * E)
    wmat = jnp.transpose(W, (0, 2, 1)).reshape(C_OUT, K * C_IN).astype(jnp.bfloat16)
    feT = _tc_transpose(fe2)
    fg = _sc_gather(feT, idx_flat).reshape(K, E, C_IN)
    raw, stats = _tc_conv(fg, wmat)
    out = _tc_norm(raw, stats)
    return out[None]


# speedup vs baseline: 1.2212x; 1.2212x over previous
import functools

import jax
import jax.numpy as jnp
from jax import lax
from jax.experimental import pallas as pl
from jax.experimental.pallas import tpu as pltpu
from jax.experimental.pallas import tpu_sc as plsc

B, C_IN, C_OUT, E, K = 1, 64, 64, 320000, 5
EPS = 1e-05

NC, NS = 2, 16
NW = NC * NS
ROWS_PER_W = K * E // NW
CHUNK = 128
NB = 6
NFULL = ROWS_PER_W // CHUNK
NGROUPS = NFULL // NB
TAIL = ROWS_PER_W - NFULL * CHUNK


def _sc_gather(feT, idx_flat):
    mesh = plsc.VectorSubcoreMesh(core_axis_name="c", subcore_axis_name="s")

    @functools.partial(
        pl.kernel,
        out_type=jax.ShapeDtypeStruct((K * E, C_PK), feT.dtype),
        mesh=mesh,
        scratch_types=[
            pltpu.VMEM((NB, CHUNK), jnp.int32),
            pltpu.VMEM((NB, CHUNK, C_PK), feT.dtype),
            pltpu.SemaphoreType.DMA,
            pltpu.SemaphoreType.DMA,
        ],
        compiler_params=pltpu.CompilerParams(use_tc_tiling_on_sc=False),
    )
    def k(feT_hbm, idx_hbm, out_hbm, idx_v, rows_v, gsem, wsem):
        wid = lax.axis_index("s") * NC + lax.axis_index("c")
        base = wid * ROWS_PER_W

        def group(g, carry):
            starts = [base + (g * NB + b) * CHUNK for b in range(NB)]
            gh = []
            for b in range(NB):
                pltpu.sync_copy(idx_hbm.at[pl.ds(starts[b], CHUNK)], idx_v.at[b])
                gh.append(pltpu.async_copy(feT_hbm.at[idx_v.at[b]], rows_v.at[b], gsem))
            wh = []
            for b in range(NB):
                gh[b].wait()
                wh.append(pltpu.async_copy(rows_v.at[b], out_hbm.at[pl.ds(starts[b], CHUNK)], wsem))
            for b in range(NB):
                wh[b].wait()
            return carry

        lax.fori_loop(0, NGROUPS, group, 0, unroll=False)

        tstart = base + NFULL * CHUNK
        pltpu.sync_copy(idx_hbm.at[pl.ds(tstart, TAIL)], idx_v.at[0, pl.ds(0, TAIL)])
        pltpu.async_copy(feT_hbm.at[idx_v.at[0, pl.ds(0, TAIL)]],
                         rows_v.at[0, pl.ds(0, TAIL)], gsem).wait()
        pltpu.async_copy(rows_v.at[0, pl.ds(0, TAIL)],
                         out_hbm.at[pl.ds(tstart, TAIL)], wsem).wait()

    return k(feT, idx_flat)


ET = 1280


C_PK = C_IN // 2


def _round_bf16_bits(u):
    return (u + jnp.uint32(0x7FFF) + ((u >> 16) & jnp.uint32(1))) >> 16


def _tc_transpose(fe2):
    def body(fe_ref, o_ref):
        t = fe_ref[...].T
        u = lax.bitcast_convert_type(t, jnp.uint32)
        rb = _round_bf16_bits(u)
        y = (rb[:, C_PK:] << 16) | rb[:, :C_PK]
        o_ref[...] = lax.bitcast_convert_type(y, jnp.int32)

    return pl.pallas_call(
        body,
        grid=(E // ET,),
        in_specs=[pl.BlockSpec((C_IN, ET), lambda i: (0, i))],
        out_specs=pl.BlockSpec((ET, C_PK), lambda i: (i, 0)),
        out_shape=jax.ShapeDtypeStruct((E, C_PK), jnp.int32),
    )(fe2)


def _tc_conv(fg, wmat):
    def unpack(y):
        u = lax.bitcast_convert_type(y, jnp.uint32)
        lo = lax.bitcast_convert_type(u << 16, jnp.float32)
        hi = lax.bitcast_convert_type(u & jnp.uint32(0xFFFF0000), jnp.float32)
        return jnp.concatenate([lo, hi], axis=1)

    def body(fg_ref, w_ref, raw_ref, stats_ref):
        i = pl.program_id(0)
        f0 = unpack(fg_ref[0])
        f1 = unpack(fg_ref[1])
        f2 = unpack(fg_ref[2])
        f3 = unpack(fg_ref[3])
        f4 = unpack(fg_ref[4])
        fn = jnp.concatenate(
            [f0, f1 + f3, f2 + f4, jnp.abs(f1 - f3), jnp.abs(f2 - f4)],
            axis=1).astype(jnp.bfloat16)
        out = lax.dot_general(w_ref[...], fn, (((1,), (1,)), ((), ())),
                              preferred_element_type=jnp.float32)
        raw_ref[...] = out.astype(jnp.bfloat16)

        @pl.when(i == 0)
        def _init():
            stats_ref[...] = jnp.zeros_like(stats_ref)

        s = jnp.sum(out, axis=1, keepdims=True)
        sq = jnp.sum(out * out, axis=1, keepdims=True)
        stats_ref[...] += jnp.concatenate([s, sq], axis=1)

    return pl.pallas_call(
        body,
        grid=(E // ET,),
        in_specs=[pl.BlockSpec((K, ET, C_PK), lambda i: (0, i, 0)),
                  pl.BlockSpec((C_OUT, K * C_IN), lambda i: (0, 0))],
        out_specs=[pl.BlockSpec((C_OUT, ET), lambda i: (0, i)),
                   pl.BlockSpec((C_OUT, 2), lambda i: (0, 0))],
        out_shape=[jax.ShapeDtypeStruct((C_OUT, E), jnp.bfloat16),
                   jax.ShapeDtypeStruct((C_OUT, 2), jnp.float32)],
    )(fg, wmat)


def _tc_norm(raw, stats):
    inv_e = 1.0 / E

    def body(raw_ref, st_ref, o_ref):
        st = st_ref[...]
        mean = st[:, 0:1] * inv_e
        ex2 = st[:, 1:2] * inv_e
        var = jnp.maximum(ex2 - mean * mean, 0.0)
        rstd = 1.0 / (jnp.sqrt(var) + EPS)
        raw = raw_ref[...].astype(jnp.float32)
        o_ref[...] = jnp.maximum((raw - mean) * rstd, 0.0)

    return pl.pallas_call(
        body,
        grid=(E // ET,),
        in_specs=[pl.BlockSpec((C_OUT, ET), lambda i: (0, i)),
                  pl.BlockSpec((C_OUT, 2), lambda i: (0, 0))],
        out_specs=pl.BlockSpec((C_OUT, ET), lambda i: (0, i)),
        out_shape=jax.ShapeDtypeStruct((C_OUT, E), jnp.float32),
    )(raw, stats)


def kernel(fe, gemm_edges, W, b):
    del b
    fe2 = fe[0]
    idx_flat = gemm_edges[0].T.reshape(K * E)
    wmat = jnp.transpose(W, (0, 2, 1)).reshape(C_OUT, K * C_IN).astype(jnp.bfloat16)
    feT = _tc_transpose(fe2)
    fg = _sc_gather(feT, idx_flat).reshape(K, E, C_PK)
    raw, stats = _tc_conv(fg, wmat)
    out = _tc_norm(raw, stats)
    return out[None]


# speedup vs baseline: 1.7683x; 1.4480x over previous
import functools

import jax
import jax.numpy as jnp
from jax import lax
from jax.experimental import pallas as pl
from jax.experimental.pallas import tpu as pltpu
from jax.experimental.pallas import tpu_sc as plsc

B, C_IN, C_OUT, E, K = 1, 64, 64, 320000, 5
EPS = 1e-05
C_PK = C_IN // 2
HC = C_IN // 2

NC, NS = 2, 16
NW = NC * NS
ROWS_PER_W = K * E // NW
CHUNK = 128
NB = 5
GROUP = NB * CHUNK
NG = ROWS_PER_W // GROUP
TAIL = ROWS_PER_W - NG * GROUP


def _sc_gather(feT, idx_flat):
    mesh = plsc.VectorSubcoreMesh(core_axis_name="c", subcore_axis_name="s")

    @functools.partial(
        pl.kernel,
        out_type=jax.ShapeDtypeStruct((K * E, C_PK), feT.dtype),
        mesh=mesh,
        scratch_types=[
            pltpu.VMEM((ROWS_PER_W,), jnp.int32),
            pltpu.VMEM((2, GROUP, C_PK), feT.dtype),
            pltpu.SemaphoreType.DMA,
            pltpu.SemaphoreType.DMA,
        ],
        compiler_params=pltpu.CompilerParams(use_tc_tiling_on_sc=False),
    )
    def k(feT_hbm, idx_hbm, out_hbm, idx_v, rows_v, gsem, wsem):
        wid = lax.axis_index("s") * NC + lax.axis_index("c")
        base = wid * ROWS_PER_W
        pltpu.sync_copy(idx_hbm.at[pl.ds(base, ROWS_PER_W)], idx_v)

        def group(t, carry):
            for p in range(2):
                g = 2 * t + p
                start = base + g * GROUP

                @pl.when(t > 0)
                def _drain():
                    pltpu.make_async_copy(
                        rows_v.at[p], out_hbm.at[pl.ds(base, GROUP)], wsem).wait()

                gh = []
                for b in range(NB):
                    isl = idx_v.at[pl.ds(g * GROUP + b * CHUNK, CHUNK)]
                    gh.append(pltpu.async_copy(
                        feT_hbm.at[isl], rows_v.at[p, pl.ds(b * CHUNK, CHUNK)], gsem))
                for h in gh:
                    h.wait()
                pltpu.async_copy(rows_v.at[p], out_hbm.at[pl.ds(start, GROUP)], wsem)
            return carry

        lax.fori_loop(0, NG // 2, group, 0, unroll=False)
        for p in range(2):
            pltpu.make_async_copy(
                rows_v.at[p], out_hbm.at[pl.ds(base, GROUP)], wsem).wait()

        tstart = base + NG * GROUP
        isl = idx_v.at[pl.ds(NG * GROUP, TAIL)]
        pltpu.async_copy(feT_hbm.at[isl], rows_v.at[0, pl.ds(0, TAIL)], gsem).wait()
        pltpu.async_copy(rows_v.at[0, pl.ds(0, TAIL)],
                         out_hbm.at[pl.ds(tstart, TAIL)], wsem).wait()

    return k(feT, idx_flat)


ET = 2560


def _round_bf16_bits(u):
    return (u + jnp.uint32(0x7FFF) + ((u >> 16) & jnp.uint32(1))) >> 16


def _tc_transpose(fe2):
    def body(fe_ref, o_ref):
        u = lax.bitcast_convert_type(fe_ref[...], jnp.uint32)
        rb = _round_bf16_bits(u)
        y = (rb[HC:, :] << 16) | rb[:HC, :]
        o_ref[...] = lax.bitcast_convert_type(y.T, jnp.int32)

    return pl.pallas_call(
        body,
        grid=(E // ET,),
        in_specs=[pl.BlockSpec((C_IN, ET), lambda i: (0, i))],
        out_specs=pl.BlockSpec((ET, C_PK), lambda i: (i, 0)),
        out_shape=jax.ShapeDtypeStruct((E, C_PK), jnp.int32),
    )(fe2)


def _tc_conv(fg, wx):
    def body(fg_ref, w_ref, raw_ref, stats_ref):
        i = pl.program_id(0)
        u = lax.bitcast_convert_type(fg_ref[...], jnp.uint32)
        lo = lax.bitcast_convert_type(u << 16, jnp.float32)
        hi = lax.bitcast_convert_type(u & jnp.uint32(0xFFFF0000), jnp.float32)
        d_lo = jnp.abs(lo[:, C_PK:3 * C_PK] - lo[:, 3 * C_PK:])
        d_hi = jnp.abs(hi[:, C_PK:3 * C_PK] - hi[:, 3 * C_PK:])
        fnx = jnp.concatenate([lo, hi, d_lo, d_hi], axis=1).astype(jnp.bfloat16)
        out = lax.dot_general(w_ref[...], fnx, (((1,), (1,)), ((), ())),
                              preferred_element_type=jnp.float32)
        raw_ref[...] = out.astype(jnp.bfloat16)

        @pl.when(i == 0)
        def _init():
            stats_ref[...] = jnp.zeros_like(stats_ref)

        s = jnp.sum(out, axis=1, keepdims=True)
        sq = jnp.sum(out * out, axis=1, keepdims=True)
        stats_ref[...] += jnp.concatenate([s, sq], axis=1)

    return pl.pallas_call(
        body,
        grid=(E // ET,),
        in_specs=[pl.BlockSpec((ET, K * C_PK), lambda i: (i, 0)),
                  pl.BlockSpec((C_OUT, 7 * C_IN), lambda i: (0, 0))],
        out_specs=[pl.BlockSpec((C_OUT, ET), lambda i: (0, i)),
                   pl.BlockSpec((C_OUT, 2), lambda i: (0, 0))],
        out_shape=[jax.ShapeDtypeStruct((C_OUT, E), jnp.bfloat16),
                   jax.ShapeDtypeStruct((C_OUT, 2), jnp.float32)],
    )(fg, wx)


def _tc_norm(raw, stats):
    inv_e = 1.0 / E

    def body(raw_ref, st_ref, o_ref):
        st = st_ref[...]
        mean = st[:, 0:1] * inv_e
        ex2 = st[:, 1:2] * inv_e
        var = jnp.maximum(ex2 - mean * mean, 0.0)
        rstd = 1.0 / (jnp.sqrt(var) + EPS)
        raw = raw_ref[...].astype(jnp.float32)
        o_ref[...] = jnp.maximum((raw - mean) * rstd, 0.0)

    return pl.pallas_call(
        body,
        grid=(E // ET,),
        in_specs=[pl.BlockSpec((C_OUT, ET), lambda i: (0, i)),
                  pl.BlockSpec((C_OUT, 2), lambda i: (0, 0))],
        out_specs=pl.BlockSpec((C_OUT, ET), lambda i: (0, i)),
        out_shape=jax.ShapeDtypeStruct((C_OUT, E), jnp.float32),
    )(raw, stats)


def _build_wx(W):
    lk = [0, 1, 2, 1, 2]
    lo_cols = [W[:, :HC, lk[k]] for k in range(K)]
    hi_cols = [W[:, HC:, lk[k]] for k in range(K)]
    d_lo = [W[:, :HC, 3], W[:, :HC, 4]]
    d_hi = [W[:, HC:, 3], W[:, HC:, 4]]
    return jnp.concatenate(lo_cols + hi_cols + d_lo + d_hi,
                           axis=1).astype(jnp.bfloat16)


def kernel(fe, gemm_edges, W, b):
    del b
    fe2 = fe[0]
    idx_flat = gemm_edges[0].reshape(K * E)
    wx = _build_wx(W)
    feT = _tc_transpose(fe2)
    fg = _sc_gather(feT, idx_flat).reshape(E, K * C_PK)
    raw, stats = _tc_conv(fg, wx)
    out = _tc_norm(raw, stats)
    return out[None]


# speedup vs baseline: 3.5693x; 2.0185x over previous
import functools

import jax
import jax.numpy as jnp
from jax import lax
from jax.experimental import pallas as pl
from jax.experimental.pallas import tpu as pltpu
from jax.experimental.pallas import tpu_sc as plsc

B, C_IN, C_OUT, E, K = 1, 64, 64, 320000, 5
EPS = 1e-05
C_PK = C_IN // 2
HC = C_IN // 2

ET = 2560
SL = ET // 4
NBLOCK = E // ET
UR = 1280
PH = UR // 4
NUNIT = NBLOCK * K * 2
NW = 32
NC = 2
CHUNK = 128
NCH = UR // CHUNK
NPIECE = 14


def _sc_gather(feT, idx_km):
    mesh = plsc.VectorSubcoreMesh(core_axis_name="c", subcore_axis_name="s")

    @functools.partial(
        pl.kernel,
        out_type=jax.ShapeDtypeStruct((K * E, C_PK), jnp.int32),
        mesh=mesh,
        scratch_types=[
            pltpu.VMEM((2, UR), jnp.int32),
            pltpu.VMEM((2, UR), jnp.int32),
            pltpu.VMEM((2, UR, C_PK), jnp.int32),
            pltpu.SemaphoreType.DMA,
            pltpu.SemaphoreType.DMA,
            pltpu.SemaphoreType.DMA,
        ],
        compiler_params=pltpu.CompilerParams(use_tc_tiling_on_sc=False,
                                             needs_layout_passes=False),
    )
    def k(feT_hbm, idx_hbm, out_hbm, idx_raw, idx_v, rows_v, isem, gsem, wsem):
        wid = lax.axis_index("s") * NC + lax.axis_index("c")
        nfull = NUNIT // NW
        rem = NUNIT - nfull * NW
        count = jnp.where(wid < rem, nfull + 1, nfull)

        def load_idx(u, p):
            s, h = u // 2, lax.rem(u, 2)
            i, kk = s // K, lax.rem(s, K)
            base = kk * E + i * ET + h * PH
            for j in range(4):
                pltpu.async_copy(
                    idx_hbm.at[pl.ds(base + j * SL, PH)],
                    idx_raw.at[p, pl.ds(j * PH, PH)], isem)

        def wait_idx(p):
            for j in range(4):
                pltpu.make_async_copy(
                    idx_hbm.at[pl.ds(0, PH)],
                    idx_raw.at[p, pl.ds(0, PH)], isem).wait()

        load_idx(wid, 0)

        def unit(t, carry):
            p = lax.rem(t, 2)
            u = wid + NW * t

            @pl.when(t < count)
            def _work():
                wait_idx(p)

                @pl.when(t + 1 < count)
                def _pf():
                    load_idx(u + NW, 1 - p)

                iota = lax.iota(jnp.int32, 16)
                for bq in range(UR // 16):
                    rv = iota + (16 * bq)
                    pos = (rv & 3) * PH + (rv >> 2)
                    vals = plsc.load_gather(idx_raw.at[p], [pos])
                    idx_v[p, pl.ds(16 * bq, 16)] = vals

                @pl.when(t >= 2)
                def _drain():
                    pltpu.make_async_copy(
                        rows_v.at[p], out_hbm.at[pl.ds(0, UR)], wsem).wait()

                gh = []
                for c in range(NCH):
                    isl = idx_v.at[p, pl.ds(c * CHUNK, CHUNK)]
                    gh.append(pltpu.async_copy(
                        feT_hbm.at[isl], rows_v.at[p, pl.ds(c * CHUNK, CHUNK)],
                        gsem))
                for h in gh:
                    h.wait()
                pltpu.async_copy(rows_v.at[p], out_hbm.at[pl.ds(u * UR, UR)], wsem)

            return carry

        lax.fori_loop(0, nfull + 1, unit, 0, unroll=False)

        for p in range(2):
            @pl.when(count >= 2 - p)
            def _final_drain():
                pltpu.make_async_copy(
                    rows_v.at[p], out_hbm.at[pl.ds(0, UR)], wsem).wait()

    return k(feT, idx_km)


ET1 = 2560


def _round_bf16_bits(u):
    return (u + jnp.uint32(0x7FFF) + ((u >> 16) & jnp.uint32(1))) >> 16


def _tc_transpose(fe2):
    def body(fe_ref, o_ref):
        u = lax.bitcast_convert_type(fe_ref[...], jnp.uint32)
        rb = _round_bf16_bits(u)
        y = (rb[HC:, :] << 16) | rb[:HC, :]
        t = lax.bitcast_convert_type(y.T, jnp.int32)
        t4 = t.reshape(ET1 // 4, 4, C_PK)
        o_ref[...] = jnp.concatenate([t4[:, j, :] for j in range(4)], axis=1)

    return pl.pallas_call(
        body,
        grid=(E // ET1,),
        in_specs=[pl.BlockSpec((C_IN, ET1), lambda i: (0, i))],
        out_specs=pl.BlockSpec((ET1 // 4, 128), lambda i: (i, 0)),
        out_shape=jax.ShapeDtypeStruct((E // 4, 128), jnp.int32),
    )(fe2)


def _unpack(y):
    u = lax.bitcast_convert_type(y, jnp.uint32)
    lo = lax.bitcast_convert_type(u << 16, jnp.float32)
    hi = lax.bitcast_convert_type(u & jnp.uint32(0xFFFF0000), jnp.float32)
    return lo, hi


def _tc_conv(fgq, wx4):
    def body(fg_ref, w_ref, raw_ref, stats_ref):
        i = pl.program_id(0)
        x = fg_ref[...]
        los, his = [], []
        for kk in range(K):
            lo, hi = _unpack(x[kk * SL:(kk + 1) * SL, :])
            los.append(lo)
            his.append(hi)
        d13lo = jnp.abs(los[1] - los[3])
        d24lo = jnp.abs(los[2] - los[4])
        d13hi = jnp.abs(his[1] - his[3])
        d24hi = jnp.abs(his[2] - his[4])
        pieces = los + his + [d13lo, d24lo, d13hi, d24hi]
        fnx = jnp.concatenate(pieces, axis=1).astype(jnp.bfloat16)
        out4 = lax.dot_general(w_ref[...], fnx, (((1,), (1,)), ((), ())),
                               preferred_element_type=jnp.float32)
        raw_ref[...] = out4.astype(jnp.bfloat16)

        @pl.when(i == 0)
        def _init():
            stats_ref[...] = jnp.zeros_like(stats_ref)

        s = jnp.sum(out4, axis=1, keepdims=True)
        sq = jnp.sum(out4 * out4, axis=1, keepdims=True)
        stats_ref[...] += jnp.concatenate([s, sq], axis=1)

    return pl.pallas_call(
        body,
        grid=(NBLOCK,),
        in_specs=[pl.BlockSpec((K * SL, 128), lambda i: (i, 0)),
                  pl.BlockSpec((4 * C_OUT, NPIECE * 128), lambda i: (0, 0))],
        out_specs=[pl.BlockSpec((4 * C_OUT, SL), lambda i: (0, i)),
                   pl.BlockSpec((4 * C_OUT, 2), lambda i: (0, 0))],
        out_shape=[jax.ShapeDtypeStruct((4 * C_OUT, E // 4), jnp.bfloat16),
                   jax.ShapeDtypeStruct((4 * C_OUT, 2), jnp.float32)],
    )(fgq, wx4)


def _tc_norm(raw, stats):
    inv_e = 1.0 / E

    def body(raw_ref, st_ref, o_ref):
        st4 = st_ref[...]
        st = (st4[0:C_OUT] + st4[C_OUT:2 * C_OUT]
              + st4[2 * C_OUT:3 * C_OUT] + st4[3 * C_OUT:])
        mean = st[:, 0:1] * inv_e
        ex2 = st[:, 1:2] * inv_e
        var = jnp.maximum(ex2 - mean * mean, 0.0)
        rstd = 1.0 / (jnp.sqrt(var) + EPS)
        mean4 = jnp.concatenate([mean] * 4, axis=0)
        rstd4 = jnp.concatenate([rstd] * 4, axis=0)
        x = raw_ref[...].astype(jnp.float32)
        y = jnp.maximum((x - mean4) * rstd4, 0.0)
        for j in range(4):
            o_ref[:, j * SL:(j + 1) * SL] = y[j * C_OUT:(j + 1) * C_OUT, :]

    return pl.pallas_call(
        body,
        grid=(NBLOCK,),
        in_specs=[pl.BlockSpec((4 * C_OUT, SL), lambda i: (0, i)),
                  pl.BlockSpec((4 * C_OUT, 2), lambda i: (0, 0))],
        out_specs=pl.BlockSpec((C_OUT, ET), lambda i: (0, i)),
        out_shape=jax.ShapeDtypeStruct((C_OUT, E), jnp.float32),
    )(raw, stats)


def _build_wx4(W):
    lk = [0, 1, 2, 1, 2]
    lo_cols = [W[:, :HC, lk[k]] for k in range(K)]
    hi_cols = [W[:, HC:, lk[k]] for k in range(K)]
    d_cols = [W[:, :HC, 3], W[:, :HC, 4], W[:, HC:, 3], W[:, HC:, 4]]
    wx = jnp.stack(lo_cols + hi_cols + d_cols, axis=1)
    rows = []
    for j in range(4):
        blk = jnp.zeros((C_OUT, NPIECE, 4, HC), jnp.float32)
        blk = blk.at[:, :, j, :].set(wx)
        rows.append(blk.reshape(C_OUT, NPIECE * 128))
    return jnp.concatenate(rows, axis=0).astype(jnp.bfloat16)


def kernel(fe, gemm_edges, W, b):
    del b
    fe2 = fe[0]
    idx_km = gemm_edges[0].T.reshape(K * E)
    wx4 = _build_wx4(W)
    feT4 = _tc_transpose(fe2)
    fg = _sc_gather(feT4.reshape(E, C_PK), idx_km)
    fgq = fg.reshape(K * E // 4, 128)
    raw, stats = _tc_conv(fgq, wx4)
    out = _tc_norm(raw, stats)
    return out[None]


# speedup vs baseline: 3.5813x; 1.0034x over previous
import functools

import jax
import jax.numpy as jnp
from jax import lax
from jax.experimental import pallas as pl
from jax.experimental.pallas import tpu as pltpu
from jax.experimental.pallas import tpu_sc as plsc

B, C_IN, C_OUT, E, K = 1, 64, 64, 320000, 5
EPS = 1e-05
C_PK = C_IN // 2
HC = C_IN // 2

ET = 2560
SL = ET // 4
NBLOCK = E // ET
UR = 1280
PH = UR // 4
NUNIT = NBLOCK * K * 2
NW = 32
NC = 2
CHUNK = 128
NCH = UR // CHUNK
NPIECE = 14


def _sc_gather(feT, idx_km):
    mesh = plsc.VectorSubcoreMesh(core_axis_name="c", subcore_axis_name="s")

    @functools.partial(
        pl.kernel,
        out_type=jax.ShapeDtypeStruct((K * E, C_PK), jnp.int32),
        mesh=mesh,
        scratch_types=[
            pltpu.VMEM((2, UR), jnp.int32),
            pltpu.VMEM((2, UR), jnp.int32),
            pltpu.VMEM((2, UR, C_PK), jnp.int32),
            pltpu.SemaphoreType.DMA,
            pltpu.SemaphoreType.DMA,
            pltpu.SemaphoreType.DMA,
        ],
        compiler_params=pltpu.CompilerParams(use_tc_tiling_on_sc=False,
                                             needs_layout_passes=False),
    )
    def k(feT_hbm, idx_hbm, out_hbm, idx_raw, idx_v, rows_v, isem, gsem, wsem):
        wid = lax.axis_index("s") * NC + lax.axis_index("c")
        nfull = NUNIT // NW
        rem = NUNIT - nfull * NW
        count = jnp.where(wid < rem, nfull + 1, nfull)

        def load_idx(u, p):
            s, h = u // 2, lax.rem(u, 2)
            i, kk = s // K, lax.rem(s, K)
            base = kk * E + i * ET + h * PH
            for j in range(4):
                pltpu.async_copy(
                    idx_hbm.at[pl.ds(base + j * SL, PH)],
                    idx_raw.at[p, pl.ds(j * PH, PH)], isem)

        def wait_idx(p):
            for j in range(4):
                pltpu.make_async_copy(
                    idx_hbm.at[pl.ds(0, PH)],
                    idx_raw.at[p, pl.ds(0, PH)], isem).wait()

        load_idx(wid, 0)

        def unit(t, carry):
            p = lax.rem(t, 2)
            u = wid + NW * t

            @pl.when(t < count)
            def _work():
                wait_idx(p)

                @pl.when(t + 1 < count)
                def _pf():
                    load_idx(u + NW, 1 - p)

                iota = lax.iota(jnp.int32, 16)
                for bq in range(UR // 16):
                    rv = iota + (16 * bq)
                    pos = (rv & 3) * PH + (rv >> 2)
                    vals = plsc.load_gather(idx_raw.at[p], [pos])
                    idx_v[p, pl.ds(16 * bq, 16)] = vals

                @pl.when(t >= 2)
                def _drain():
                    pltpu.make_async_copy(
                        rows_v.at[p], out_hbm.at[pl.ds(0, UR)], wsem).wait()

                gh = []
                for c in range(NCH):
                    isl = idx_v.at[p, pl.ds(c * CHUNK, CHUNK)]
                    gh.append(pltpu.async_copy(
                        feT_hbm.at[isl], rows_v.at[p, pl.ds(c * CHUNK, CHUNK)],
                        gsem))
                for h in gh:
                    h.wait()
                pltpu.async_copy(rows_v.at[p], out_hbm.at[pl.ds(u * UR, UR)], wsem)

            return carry

        lax.fori_loop(0, nfull + 1, unit, 0, unroll=False)

        for p in range(2):
            @pl.when(count >= 2 - p)
            def _final_drain():
                pltpu.make_async_copy(
                    rows_v.at[p], out_hbm.at[pl.ds(0, UR)], wsem).wait()

    return k(feT, idx_km)


ET1 = 2560


def _round_bf16_bits(u):
    return (u + jnp.uint32(0x7FFF) + ((u >> 16) & jnp.uint32(1))) >> 16


def _tc_transpose(fe2):
    def body(fe_ref, o_ref):
        u = lax.bitcast_convert_type(fe_ref[...], jnp.uint32)
        rb = _round_bf16_bits(u)
        y = (rb[HC:, :] << 16) | rb[:HC, :]
        t = lax.bitcast_convert_type(y.T, jnp.int32)
        t4 = t.reshape(ET1 // 4, 4, C_PK)
        for j in range(4):
            o_ref[:, C_PK * j:C_PK * (j + 1)] = t4[:, j, :]

    return pl.pallas_call(
        body,
        grid=(E // ET1,),
        in_specs=[pl.BlockSpec((C_IN, ET1), lambda i: (0, i))],
        out_specs=pl.BlockSpec((ET1 // 4, 128), lambda i: (i, 0)),
        out_shape=jax.ShapeDtypeStruct((E // 4, 128), jnp.int32),
    )(fe2)


def _unpack(y):
    u = lax.bitcast_convert_type(y, jnp.uint32)
    lo = lax.bitcast_convert_type(u << 16, jnp.float32)
    hi = lax.bitcast_convert_type(u & jnp.uint32(0xFFFF0000), jnp.float32)
    return lo, hi


def _tc_conv(fgq, wx4):
    def body(fg_ref, w_ref, raw_ref, stats_ref):
        i = pl.program_id(0)
        x = fg_ref[...]
        los, his = [], []
        for kk in range(K):
            lo, hi = _unpack(x[kk * SL:(kk + 1) * SL, :])
            los.append(lo)
            his.append(hi)
        d13lo = jnp.abs(los[1] - los[3])
        d24lo = jnp.abs(los[2] - los[4])
        d13hi = jnp.abs(his[1] - his[3])
        d24hi = jnp.abs(his[2] - his[4])
        pieces = los + his + [d13lo, d24lo, d13hi, d24hi]
        fnx = jnp.concatenate(pieces, axis=1).astype(jnp.bfloat16)
        out4 = lax.dot_general(w_ref[...], fnx, (((1,), (1,)), ((), ())),
                               preferred_element_type=jnp.float32)
        raw_ref[...] = out4.astype(jnp.bfloat16)

        @pl.when(i == 0)
        def _init():
            stats_ref[...] = jnp.zeros_like(stats_ref)

        s = jnp.sum(out4, axis=1, keepdims=True)
        sq = jnp.sum(out4 * out4, axis=1, keepdims=True)
        stats_ref[...] += jnp.concatenate([s, sq], axis=1)

    return pl.pallas_call(
        body,
        grid=(NBLOCK,),
        in_specs=[pl.BlockSpec((K * SL, 128), lambda i: (i, 0)),
                  pl.BlockSpec((4 * C_OUT, NPIECE * 128), lambda i: (0, 0))],
        out_specs=[pl.BlockSpec((4 * C_OUT, SL), lambda i: (0, i)),
                   pl.BlockSpec((4 * C_OUT, 2), lambda i: (0, 0))],
        out_shape=[jax.ShapeDtypeStruct((4 * C_OUT, E // 4), jnp.bfloat16),
                   jax.ShapeDtypeStruct((4 * C_OUT, 2), jnp.float32)],
    )(fgq, wx4)


def _tc_norm(raw, stats):
    inv_e = 1.0 / E

    def body(raw_ref, st_ref, o_ref):
        st4 = st_ref[...]
        st = (st4[0:C_OUT] + st4[C_OUT:2 * C_OUT]
              + st4[2 * C_OUT:3 * C_OUT] + st4[3 * C_OUT:])
        mean = st[:, 0:1] * inv_e
        ex2 = st[:, 1:2] * inv_e
        var = jnp.maximum(ex2 - mean * mean, 0.0)
        rstd = 1.0 / (jnp.sqrt(var) + EPS)
        mean4 = jnp.concatenate([mean] * 4, axis=0)
        rstd4 = jnp.concatenate([rstd] * 4, axis=0)
        x = raw_ref[...].astype(jnp.float32)
        y = jnp.maximum((x - mean4) * rstd4, 0.0)
        for j in range(4):
            o_ref[:, j * SL:(j + 1) * SL] = y[j * C_OUT:(j + 1) * C_OUT, :]

    return pl.pallas_call(
        body,
        grid=(NBLOCK,),
        in_specs=[pl.BlockSpec((4 * C_OUT, SL), lambda i: (0, i)),
                  pl.BlockSpec((4 * C_OUT, 2), lambda i: (0, 0))],
        out_specs=pl.BlockSpec((C_OUT, ET), lambda i: (0, i)),
        out_shape=jax.ShapeDtypeStruct((C_OUT, E), jnp.float32),
    )(raw, stats)


def _build_wx4(W):
    lk = [0, 1, 2, 1, 2]
    lo_cols = [W[:, :HC, lk[k]] for k in range(K)]
    hi_cols = [W[:, HC:, lk[k]] for k in range(K)]
    d_cols = [W[:, :HC, 3], W[:, :HC, 4], W[:, HC:, 3], W[:, HC:, 4]]
    wx = jnp.stack(lo_cols + hi_cols + d_cols, axis=1)
    rows = []
    for j in range(4):
        blk = jnp.zeros((C_OUT, NPIECE, 4, HC), jnp.float32)
        blk = blk.at[:, :, j, :].set(wx)
        rows.append(blk.reshape(C_OUT, NPIECE * 128))
    return jnp.concatenate(rows, axis=0).astype(jnp.bfloat16)


def kernel(fe, gemm_edges, W, b):
    del b
    fe2 = fe[0]
    idx_km = gemm_edges[0].T.reshape(K * E)
    wx4 = _build_wx4(W)
    feT4 = _tc_transpose(fe2)
    fg = _sc_gather(feT4.reshape(E, C_PK), idx_km)
    fgq = fg.reshape(K * E // 4, 128)
    raw, stats = _tc_conv(fgq, wx4)
    out = _tc_norm(raw, stats)
    return out[None]
